# baseline (speedup 1.0000x reference)
import jax
import jax.numpy as jnp
from jax import lax
from jax.experimental import pallas as pl
from jax.experimental.pallas import tpu as pltpu
from jax.experimental.pallas import tpu_sc as plsc

N = 10000
E = 320000
D = 128

NC = 2
NS = 16
EPC = E // NC
EPW = EPC // NS
G = 80
NB = EPW // G
NZB = N // G


def _sc_segment_sum(src, dst, x):
    mesh = plsc.VectorSubcoreMesh(core_axis_name="c", subcore_axis_name="s")

    def body(src_hbm, dst_hbm, x_hbm, out_hbm,
             acc, rows0, rows1, sidx, didx, sem0, sem1, ssem0, ssem1):
        c = lax.axis_index("c")
        s = lax.axis_index("s")

        zvec = jnp.zeros((16,), jnp.float32)

        @pl.loop(0, G)
        def _(i):
            @pl.loop(0, D // 16)
            def _(j):
                rows0[i, pl.ds(j * 16, 16)] = zvec

        @pl.loop(s, NZB, step=NS)
        def _(k):
            pltpu.sync_copy(rows0, acc.at[pl.ds(k * G, G)])

        plsc.subcore_barrier()

        base = (c * NS + s) * EPW
        pltpu.sync_copy(src_hbm.at[pl.ds(base, EPW)], sidx)
        pltpu.sync_copy(dst_hbm.at[pl.ds(base, EPW)], didx)

        def start_gather(b, rows, sem):
            pltpu.async_copy(x_hbm.at[sidx.at[pl.ds(b * G, G)]], rows, sem)

        def wait_gather(b, rows, sem):
            pltpu.make_async_copy(x_hbm.at[sidx.at[pl.ds(b * G, G)]],
                                  rows, sem).wait()

        def start_scatter(b, rows, sem):
            pltpu.async_copy(rows, acc.at[didx.at[pl.ds(b * G, G)]], sem,
                             add=True)

        def wait_scatter(b, rows, sem):
            pltpu.make_async_copy(rows, acc.at[didx.at[pl.ds(b * G, G)]],
                                  sem).wait()

        NP = (NB - 1) // 2
        start_gather(0, rows0, sem0)
        start_gather(1, rows1, sem1)

        @pl.loop(0, NP)
        def _(p):
            wait_gather(2 * p, rows0, sem0)
            start_scatter(2 * p, rows0, ssem0)
            wait_gather(2 * p + 1, rows1, sem1)
            start_scatter(2 * p + 1, rows1, ssem1)
            wait_scatter(2 * p, rows0, ssem0)
            start_gather(2 * p + 2, rows0, sem0)
            wait_scatter(2 * p + 1, rows1, ssem1)

            @pl.when(p < NP - 1)
            def _():
                start_gather(2 * p + 3, rows1, sem1)

        wait_gather(NB - 1, rows0, sem0)
        start_scatter(NB - 1, rows0, ssem0)
        wait_scatter(NB - 1, rows0, ssem0)

        plsc.subcore_barrier()

        @pl.loop(s, NZB, step=NS)
        def _(k):
            pltpu.sync_copy(acc.at[pl.ds(k * G, G)], rows0)
            pltpu.sync_copy(rows0, out_hbm.at[c].at[pl.ds(k * G, G)])

    kern = pl.kernel(
        body,
        out_type=jax.ShapeDtypeStruct((NC, N, D), jnp.float32),
        mesh=mesh,
        scratch_types=[
            pltpu.VMEM_SHARED((N, D), jnp.float32),
            pltpu.VMEM((G, D), jnp.float32),
            pltpu.VMEM((G, D), jnp.float32),
            pltpu.VMEM((EPW,), jnp.int32),
            pltpu.VMEM((EPW,), jnp.int32),
            pltpu.SemaphoreType.DMA,
            pltpu.SemaphoreType.DMA,
            pltpu.SemaphoreType.DMA,
            pltpu.SemaphoreType.DMA,
        ],
    )
    return kern(src, dst, x)


def _tc_finish(partials, x, w, gamma, beta):
    B = 1000

    def body(p_ref, x_ref, w_ref, g_ref, b_ref, o_ref):
        u = p_ref[0] + p_ref[1]
        h = lax.dot_general(u, w_ref[...], (((1,), (1,)), ((), ())),
                            preferred_element_type=jnp.float32)
        h = jnp.maximum(h, 0.0) + x_ref[...]
        mean = jnp.mean(h, axis=1, keepdims=True)
        cent = h - mean
        var = jnp.mean(cent * cent, axis=1, keepdims=True)
        o_ref[...] = cent * lax.rsqrt(var + 1e-5) * g_ref[...] + b_ref[...]

    return pl.pallas_call(
        body,
        grid=(N // B,),
        in_specs=[
            pl.BlockSpec((NC, B, D), lambda i: (0, i, 0)),
            pl.BlockSpec((B, D), lambda i: (i, 0)),
            pl.BlockSpec((D, D), lambda i: (0, 0)),
            pl.BlockSpec((1, D), lambda i: (0, 0)),
            pl.BlockSpec((1, D), lambda i: (0, 0)),
        ],
        out_specs=pl.BlockSpec((B, D), lambda i: (i, 0)),
        out_shape=jax.ShapeDtypeStruct((N, D), jnp.float32),
    )(partials, x, w, gamma, beta)


@jax.jit
def kernel(x, edge_index, W, gamma, beta):
    src = edge_index[0].astype(jnp.int32)
    dst = edge_index[1].astype(jnp.int32)
    partials = _sc_segment_sum(src, dst, x)
    return _tc_finish(
        partials,
        x,
        W,
        gamma.reshape(1, D),
        beta.reshape(1, D),
    )


# speedup vs baseline: 1.2253x; 1.2253x over previous
import jax
import jax.numpy as jnp
from jax import lax
from jax.experimental import pallas as pl
from jax.experimental.pallas import tpu as pltpu
from jax.experimental.pallas import tpu_sc as plsc

N = 10000
E = 320000
D = 128

NC = 2
NS = 16
EPC = E // NC
EPW = EPC // NS
G = 80
NB = EPW // G
NZB = N // G


def _sc_segment_sum(src, dst, x):
    mesh = plsc.VectorSubcoreMesh(core_axis_name="c", subcore_axis_name="s")

    def body(src_hbm, dst_hbm, x_hbm, out_hbm,
             acc, rows0, rows1, sidx, didx, sem0, sem1):
        c = lax.axis_index("c")
        s = lax.axis_index("s")

        zvec = jnp.zeros((16,), jnp.float32)

        @pl.loop(0, G)
        def _(i):
            @pl.loop(0, D // 16)
            def _(j):
                rows0[i, pl.ds(j * 16, 16)] = zvec

        @pl.loop(s, NZB, step=NS)
        def _(k):
            pltpu.sync_copy(rows0, acc.at[pl.ds(k * G, G)])

        plsc.subcore_barrier()

        base = (c * NS + s) * EPW
        pltpu.sync_copy(src_hbm.at[pl.ds(base, EPW)], sidx)
        pltpu.sync_copy(dst_hbm.at[pl.ds(base, EPW)], didx)

        def start_gather(b, rows, sem):
            pltpu.async_copy(x_hbm.at[sidx.at[pl.ds(b * G, G)]], rows, sem)

        def wait_gather(b, rows, sem):
            pltpu.make_async_copy(x_hbm.at[sidx.at[pl.ds(b * G, G)]],
                                  rows, sem).wait()

        def finish_block(b, rows, sem):
            wait_gather(b, rows, sem)
            pltpu.sync_copy(rows, acc.at[didx.at[pl.ds(b * G, G)]], add=True)

        start_gather(0, rows0, sem0)

        @pl.loop(0, (NB - 1) // 2)
        def _(p):
            start_gather(2 * p + 1, rows1, sem1)
            finish_block(2 * p, rows0, sem0)
            start_gather(2 * p + 2, rows0, sem0)
            finish_block(2 * p + 1, rows1, sem1)

        finish_block(NB - 1, rows0, sem0)

        plsc.subcore_barrier()

        @pl.loop(s, NZB, step=NS)
        def _(k):
            pltpu.sync_copy(acc.at[pl.ds(k * G, G)], rows0)
            pltpu.sync_copy(rows0, out_hbm.at[c].at[pl.ds(k * G, G)])

    kern = pl.kernel(
        body,
        out_type=jax.ShapeDtypeStruct((NC, N, D), jnp.float32),
        mesh=mesh,
        scratch_types=[
            pltpu.VMEM_SHARED((N, D), jnp.float32),
            pltpu.VMEM((G, D), jnp.float32),
            pltpu.VMEM((G, D), jnp.float32),
            pltpu.VMEM((EPW,), jnp.int32),
            pltpu.VMEM((EPW,), jnp.int32),
            pltpu.SemaphoreType.DMA,
            pltpu.SemaphoreType.DMA,
        ],
    )
    return kern(src, dst, x)


def _tc_finish(partials, x, w, gamma, beta):
    B = 1000

    def body(p_ref, x_ref, w_ref, g_ref, b_ref, o_ref):
        u = p_ref[0] + p_ref[1]
        h = lax.dot_general(u, w_ref[...], (((1,), (1,)), ((), ())),
                            preferred_element_type=jnp.float32)
        h = jnp.maximum(h, 0.0) + x_ref[...]
        mean = jnp.mean(h, axis=1, keepdims=True)
        cent = h - mean
        var = jnp.mean(cent * cent, axis=1, keepdims=True)
        o_ref[...] = cent * lax.rsqrt(var + 1e-5) * g_ref[...] + b_ref[...]

    return pl.pallas_call(
        body,
        grid=(N // B,),
        in_specs=[
            pl.BlockSpec((NC, B, D), lambda i: (0, i, 0)),
            pl.BlockSpec((B, D), lambda i: (i, 0)),
            pl.BlockSpec((D, D), lambda i: (0, 0)),
            pl.BlockSpec((1, D), lambda i: (0, 0)),
            pl.BlockSpec((1, D), lambda i: (0, 0)),
        ],
        out_specs=pl.BlockSpec((B, D), lambda i: (i, 0)),
        out_shape=jax.ShapeDtypeStruct((N, D), jnp.float32),
    )(partials, x, w, gamma, beta)


@jax.jit
def kernel(x, edge_index, W, gamma, beta):
    src = edge_index[0].astype(jnp.int32)
    dst = edge_index[1].astype(jnp.int32)
    partials = _sc_segment_sum(src, dst, x)
    return _tc_finish(
        partials,
        x,
        W,
        gamma.reshape(1, D),
        beta.reshape(1, D),
    )


# speedup vs baseline: 1.2484x; 1.0189x over previous
import jax
import jax.numpy as jnp
from jax import lax
from jax.experimental import pallas as pl
from jax.experimental.pallas import tpu as pltpu
from jax.experimental.pallas import tpu_sc as plsc

N = 10000
E = 320000
D = 128

NC = 2
NS = 16
EPC = E // NC
EPW = EPC // NS
G = 80
NB = EPW // G
NZB = N // G


def _sc_segment_sum(src, dst, x):
    mesh = plsc.VectorSubcoreMesh(core_axis_name="c", subcore_axis_name="s")

    def body(src_hbm, dst_hbm, x_hbm, out_hbm,
             acc, rows0, rows1, sidx, didx, sem0, sem1):
        c = lax.axis_index("c")
        s = lax.axis_index("s")

        base = (c * NS + s) * EPW
        pltpu.async_copy(src_hbm.at[pl.ds(base, EPW)], sidx, sem0)
        pltpu.async_copy(dst_hbm.at[pl.ds(base, EPW)], didx, sem0)

        zvec = jnp.zeros((16,), jnp.float32)

        @pl.loop(0, G)
        def _(i):
            @pl.loop(0, D // 16)
            def _(j):
                rows0[i, pl.ds(j * 16, 16)] = zvec

        @pl.loop(s, NZB, step=NS)
        def _(k):
            pltpu.async_copy(rows0, acc.at[pl.ds(k * G, G)], sem1)

        @pl.loop(s, NZB, step=NS)
        def _(k):
            pltpu.make_async_copy(rows0, acc.at[pl.ds(k * G, G)], sem1).wait()

        pltpu.make_async_copy(src_hbm.at[pl.ds(base, EPW)], sidx, sem0).wait()
        pltpu.make_async_copy(dst_hbm.at[pl.ds(base, EPW)], didx, sem0).wait()

        plsc.subcore_barrier()

        def start_gather(b, rows, sem):
            pltpu.async_copy(x_hbm.at[sidx.at[pl.ds(b * G, G)]], rows, sem)

        def wait_gather(b, rows, sem):
            pltpu.make_async_copy(x_hbm.at[sidx.at[pl.ds(b * G, G)]],
                                  rows, sem).wait()

        def finish_block(b, rows, sem):
            wait_gather(b, rows, sem)
            pltpu.sync_copy(rows, acc.at[didx.at[pl.ds(b * G, G)]], add=True)

        start_gather(0, rows0, sem0)

        @pl.loop(0, (NB - 1) // 2)
        def _(p):
            start_gather(2 * p + 1, rows1, sem1)
            finish_block(2 * p, rows0, sem0)
            start_gather(2 * p + 2, rows0, sem0)
            finish_block(2 * p + 1, rows1, sem1)

        finish_block(NB - 1, rows0, sem0)

        plsc.subcore_barrier()

        @pl.loop(s, NZB, step=NS)
        def _(k):
            pltpu.async_copy(acc.at[pl.ds(k * G, G)],
                             out_hbm.at[c].at[pl.ds(k * G, G)], sem0)

        @pl.loop(s, NZB, step=NS)
        def _(k):
            pltpu.make_async_copy(acc.at[pl.ds(k * G, G)],
                                  out_hbm.at[c].at[pl.ds(k * G, G)],
                                  sem0).wait()

    kern = pl.kernel(
        body,
        out_type=jax.ShapeDtypeStruct((NC, N, D), jnp.float32),
        mesh=mesh,
        scratch_types=[
            pltpu.VMEM_SHARED((N, D), jnp.float32),
            pltpu.VMEM((G, D), jnp.float32),
            pltpu.VMEM((G, D), jnp.float32),
            pltpu.VMEM((EPW,), jnp.int32),
            pltpu.VMEM((EPW,), jnp.int32),
            pltpu.SemaphoreType.DMA,
            pltpu.SemaphoreType.DMA,
        ],
    )
    return kern(src, dst, x)


def _tc_finish(partials, x, w, gamma, beta):
    B = 1000

    def body(p_ref, x_ref, w_ref, g_ref, b_ref, o_ref):
        u = p_ref[0] + p_ref[1]
        h = lax.dot_general(u, w_ref[...], (((1,), (1,)), ((), ())),
                            preferred_element_type=jnp.float32)
        h = jnp.maximum(h, 0.0) + x_ref[...]
        mean = jnp.mean(h, axis=1, keepdims=True)
        cent = h - mean
        var = jnp.mean(cent * cent, axis=1, keepdims=True)
        o_ref[...] = cent * lax.rsqrt(var + 1e-5) * g_ref[...] + b_ref[...]

    return pl.pallas_call(
        body,
        grid=(N // B,),
        in_specs=[
            pl.BlockSpec((NC, B, D), lambda i: (0, i, 0)),
            pl.BlockSpec((B, D), lambda i: (i, 0)),
            pl.BlockSpec((D, D), lambda i: (0, 0)),
            pl.BlockSpec((1, D), lambda i: (0, 0)),
            pl.BlockSpec((1, D), lambda i: (0, 0)),
        ],
        out_specs=pl.BlockSpec((B, D), lambda i: (i, 0)),
        out_shape=jax.ShapeDtypeStruct((N, D), jnp.float32),
    )(partials, x, w, gamma, beta)


@jax.jit
def kernel(x, edge_index, W, gamma, beta):
    src = edge_index[0].astype(jnp.int32)
    dst = edge_index[1].astype(jnp.int32)
    partials = _sc_segment_sum(src, dst, x)
    return _tc_finish(
        partials,
        x,
        W,
        gamma.reshape(1, D),
        beta.reshape(1, D),
    )


# speedup vs baseline: 1.3572x; 1.0872x over previous
import jax
import jax.numpy as jnp
from jax import lax
from jax.experimental import pallas as pl
from jax.experimental.pallas import tpu as pltpu
from jax.experimental.pallas import tpu_sc as plsc

N = 10000
E = 320000
D = 128

NC = 2
NS = 16
EPC = E // NC
EPW = EPC // NS
G = 80
NB = EPW // G
NZB = N // G


def _sc_segment_sum(ei_flat, x):
    mesh = plsc.VectorSubcoreMesh(core_axis_name="c", subcore_axis_name="s")

    def body(ei_hbm, x_hbm, out_hbm,
             acc, rows0, rows1, sidx, didx, sem0, sem1):
        c = lax.axis_index("c")
        s = lax.axis_index("s")

        base = (c * NS + s) * EPW
        pltpu.async_copy(ei_hbm.at[pl.ds(base, EPW)], sidx, sem0)
        pltpu.async_copy(ei_hbm.at[pl.ds(E + base, EPW)], didx, sem0)

        zvec = jnp.zeros((16,), jnp.float32)

        @pl.loop(0, G)
        def _(i):
            @pl.loop(0, D // 16)
            def _(j):
                rows0[i, pl.ds(j * 16, 16)] = zvec

        @pl.loop(s, NZB, step=NS)
        def _(k):
            pltpu.async_copy(rows0, acc.at[pl.ds(k * G, G)], sem1)

        @pl.loop(s, NZB, step=NS)
        def _(k):
            pltpu.make_async_copy(rows0, acc.at[pl.ds(k * G, G)], sem1).wait()

        pltpu.make_async_copy(ei_hbm.at[pl.ds(base, EPW)], sidx, sem0).wait()
        pltpu.make_async_copy(ei_hbm.at[pl.ds(E + base, EPW)],
                              didx, sem0).wait()

        plsc.subcore_barrier()

        def start_gather(b, rows, sem):
            pltpu.async_copy(x_hbm.at[sidx.at[pl.ds(b * G, G)]], rows, sem)

        def wait_gather(b, rows, sem):
            pltpu.make_async_copy(x_hbm.at[sidx.at[pl.ds(b * G, G)]],
                                  rows, sem).wait()

        def finish_block(b, rows, sem):
            wait_gather(b, rows, sem)
            pltpu.sync_copy(rows, acc.at[didx.at[pl.ds(b * G, G)]], add=True)

        start_gather(0, rows0, sem0)

        @pl.loop(0, (NB - 1) // 2)
        def _(p):
            start_gather(2 * p + 1, rows1, sem1)
            finish_block(2 * p, rows0, sem0)
            start_gather(2 * p + 2, rows0, sem0)
            finish_block(2 * p + 1, rows1, sem1)

        finish_block(NB - 1, rows0, sem0)

        plsc.subcore_barrier()

        @pl.loop(s, NZB, step=NS)
        def _(k):
            pltpu.async_copy(acc.at[pl.ds(k * G, G)],
                             out_hbm.at[c].at[pl.ds(k * G, G)], sem0)

        @pl.loop(s, NZB, step=NS)
        def _(k):
            pltpu.make_async_copy(acc.at[pl.ds(k * G, G)],
                                  out_hbm.at[c].at[pl.ds(k * G, G)],
                                  sem0).wait()

    kern = pl.kernel(
        body,
        out_type=jax.ShapeDtypeStruct((NC, N, D), jnp.float32),
        mesh=mesh,
        scratch_types=[
            pltpu.VMEM_SHARED((N, D), jnp.float32),
            pltpu.VMEM((G, D), jnp.float32),
            pltpu.VMEM((G, D), jnp.float32),
            pltpu.VMEM((EPW,), jnp.int32),
            pltpu.VMEM((EPW,), jnp.int32),
            pltpu.SemaphoreType.DMA,
            pltpu.SemaphoreType.DMA,
        ],
    )
    return kern(ei_flat, x)


def _tc_finish(partials, x, w, gamma, beta):
    B = 2000

    def body(p_ref, x_ref, w_ref, g_ref, b_ref, o_ref):
        u = p_ref[0] + p_ref[1]
        h = lax.dot_general(u, w_ref[...], (((1,), (1,)), ((), ())),
                            preferred_element_type=jnp.float32)
        h = jnp.maximum(h, 0.0) + x_ref[...]
        mean = jnp.mean(h, axis=1, keepdims=True)
        cent = h - mean
        var = jnp.mean(cent * cent, axis=1, keepdims=True)
        o_ref[...] = cent * lax.rsqrt(var + 1e-5) * g_ref[...] + b_ref[...]

    return pl.pallas_call(
        body,
        grid=(N // B,),
        in_specs=[
            pl.BlockSpec((NC, B, D), lambda i: (0, i, 0)),
            pl.BlockSpec((B, D), lambda i: (i, 0)),
            pl.BlockSpec((D, D), lambda i: (0, 0)),
            pl.BlockSpec((1, D), lambda i: (0, 0)),
            pl.BlockSpec((1, D), lambda i: (0, 0)),
        ],
        out_specs=pl.BlockSpec((B, D), lambda i: (i, 0)),
        out_shape=jax.ShapeDtypeStruct((N, D), jnp.float32),
    )(partials, x, w, gamma, beta)


@jax.jit
def kernel(x, edge_index, W, gamma, beta):
    ei_flat = edge_index.astype(jnp.int32).reshape(2 * E)
    partials = _sc_segment_sum(ei_flat, x)
    return _tc_finish(
        partials,
        x,
        W,
        gamma.reshape(1, D),
        beta.reshape(1, D),
    )


# speedup vs baseline: 1.5658x; 1.1537x over previous
import jax
import jax.numpy as jnp
from jax import lax
from jax.experimental import pallas as pl
from jax.experimental.pallas import tpu as pltpu
from jax.experimental.pallas import tpu_sc as plsc

N = 10000
E = 320000
D = 128

NC = 2
NS = 16
EPC = E // NC
EPW = EPC // NS
G = 80
NB = EPW // G
NZB = N // G


def _sc_segment_sum(ei_flat, x):
    mesh = plsc.VectorSubcoreMesh(core_axis_name="c", subcore_axis_name="s")

    def body(ei_hbm, x_hbm, out_hbm,
             acc, rows0, rows1, rows2, sidx, didx, sem0, sem1, sem2):
        c = lax.axis_index("c")
        s = lax.axis_index("s")

        base = (c * NS + s) * EPW
        pltpu.async_copy(ei_hbm.at[pl.ds(base, EPW)], sidx, sem0)
        pltpu.async_copy(ei_hbm.at[pl.ds(E + base, EPW)], didx, sem0)

        zvec = jnp.zeros((16,), jnp.float32)

        @pl.loop(0, G)
        def _(i):
            @pl.loop(0, D // 16)
            def _(j):
                rows0[i, pl.ds(j * 16, 16)] = zvec

        @pl.loop(s, NZB, step=NS)
        def _(k):
            pltpu.async_copy(rows0, acc.at[pl.ds(k * G, G)], sem1)

        @pl.loop(s, NZB, step=NS)
        def _(k):
            pltpu.make_async_copy(rows0, acc.at[pl.ds(k * G, G)], sem1).wait()

        pltpu.make_async_copy(ei_hbm.at[pl.ds(base, EPW)], sidx, sem0).wait()
        pltpu.make_async_copy(ei_hbm.at[pl.ds(E + base, EPW)],
                              didx, sem0).wait()

        plsc.subcore_barrier()

        def start_gather(b, rows, sem):
            pltpu.async_copy(x_hbm.at[sidx.at[pl.ds(b * G, G)]], rows, sem)

        def wait_gather(b, rows, sem):
            pltpu.make_async_copy(x_hbm.at[sidx.at[pl.ds(b * G, G)]],
                                  rows, sem).wait()

        def scatter_add(b, rows):
            pltpu.sync_copy(rows, acc.at[didx.at[pl.ds(b * G, G)]], add=True)

        start_gather(0, rows0, sem0)
        start_gather(1, rows1, sem1)

        @pl.loop(0, (NB - 2) // 3)
        def _(t):
            b = 3 * t
            wait_gather(b, rows0, sem0)
            start_gather(b + 2, rows2, sem2)
            scatter_add(b, rows0)
            wait_gather(b + 1, rows1, sem1)
            start_gather(b + 3, rows0, sem0)
            scatter_add(b + 1, rows1)
            wait_gather(b + 2, rows2, sem2)
            start_gather(b + 4, rows1, sem1)
            scatter_add(b + 2, rows2)

        wait_gather(NB - 2, rows0, sem0)
        scatter_add(NB - 2, rows0)
        wait_gather(NB - 1, rows1, sem1)
        scatter_add(NB - 1, rows1)

        plsc.subcore_barrier()

        @pl.loop(s, NZB, step=NS)
        def _(k):
            pltpu.async_copy(acc.at[pl.ds(k * G, G)],
                             out_hbm.at[c].at[pl.ds(k * G, G)], sem0)

        @pl.loop(s, NZB, step=NS)
        def _(k):
            pltpu.make_async_copy(acc.at[pl.ds(k * G, G)],
                                  out_hbm.at[c].at[pl.ds(k * G, G)],
                                  sem0).wait()

    kern = pl.kernel(
        body,
        out_type=jax.ShapeDtypeStruct((NC, N, D), jnp.float32),
        mesh=mesh,
        scratch_types=[
            pltpu.VMEM_SHARED((N, D), jnp.float32),
            pltpu.VMEM((G, D), jnp.float32),
            pltpu.VMEM((G, D), jnp.float32),
            pltpu.VMEM((G, D), jnp.float32),
            pltpu.VMEM((EPW,), jnp.int32),
            pltpu.VMEM((EPW,), jnp.int32),
            pltpu.SemaphoreType.DMA,
            pltpu.SemaphoreType.DMA,
            pltpu.SemaphoreType.DMA,
        ],
    )
    return kern(ei_flat, x)


def _tc_finish(partials, x, w, gamma, beta):
    B = 2000

    def body(p_ref, x_ref, w_ref, g_ref, b_ref, o_ref):
        u = p_ref[0] + p_ref[1]
        h = lax.dot_general(u, w_ref[...], (((1,), (1,)), ((), ())),
                            preferred_element_type=jnp.float32)
        h = jnp.maximum(h, 0.0) + x_ref[...]
        mean = jnp.mean(h, axis=1, keepdims=True)
        cent = h - mean
        var = jnp.mean(cent * cent, axis=1, keepdims=True)
        o_ref[...] = cent * lax.rsqrt(var + 1e-5) * g_ref[...] + b_ref[...]

    return pl.pallas_call(
        body,
        grid=(N // B,),
        in_specs=[
            pl.BlockSpec((NC, B, D), lambda i: (0, i, 0)),
            pl.BlockSpec((B, D), lambda i: (i, 0)),
            pl.BlockSpec((D, D), lambda i: (0, 0)),
            pl.BlockSpec((1, D), lambda i: (0, 0)),
            pl.BlockSpec((1, D), lambda i: (0, 0)),
        ],
        out_specs=pl.BlockSpec((B, D), lambda i: (i, 0)),
        out_shape=jax.ShapeDtypeStruct((N, D), jnp.float32),
    )(partials, x, w, gamma, beta)


@jax.jit
def kernel(x, edge_index, W, gamma, beta):
    ei_flat = edge_index.astype(jnp.int32).reshape(2 * E)
    partials = _sc_segment_sum(ei_flat, x)
    return _tc_finish(
        partials,
        x,
        W,
        gamma.reshape(1, D),
        beta.reshape(1, D),
    )
